# baseline (speedup 1.0000x reference)
import functools

import jax
import jax.numpy as jnp
from jax import lax
from jax.experimental import pallas as pl
from jax.experimental.pallas import tpu as pltpu
from jax.experimental.pallas import tpu_sc as plsc

D_MODEL = 1024
SCALE = 32.0
NC = 2
NS = 16
NW = NC * NS
LANES = 16
C = 32
RING = 3


@functools.partial(jax.jit, static_argnums=(2,))
def _emb(idx, table, B):
    chunks = B // (NW * C)
    mesh = plsc.VectorSubcoreMesh(core_axis_name="c", subcore_axis_name="s")

    @functools.partial(
        pl.kernel,
        out_type=jax.ShapeDtypeStruct((B, D_MODEL), jnp.float32),
        mesh=mesh,
        scratch_types=(
            [pltpu.VMEM((chunks, C), jnp.int32)]
            + [pltpu.VMEM((C, D_MODEL), jnp.float32)] * RING
            + [pltpu.SemaphoreType.DMA] * (2 * RING)
        ),
    )
    def emb_kernel(idx_hbm, table_hbm, out_hbm, idx_v, *bufs_and_sems):
        bufs = bufs_and_sems[:RING]
        sis = bufs_and_sems[RING:2 * RING]
        sos = bufs_and_sems[2 * RING:]
        wid = lax.axis_index("s") * NC + lax.axis_index("c")
        base = wid * (chunks * C)
        pltpu.sync_copy(idx_hbm.at[wid], idx_v)
        pltpu.async_copy(table_hbm.at[idx_v.at[0]], bufs[0], sis[0])
        pltpu.async_copy(table_hbm.at[idx_v.at[1]], bufs[1], sis[1])

        def step(j, b, regather):
            buf, si, so = bufs[b], sis[b], sos[b]
            bg = (b + 2) % RING
            pltpu.make_async_copy(table_hbm.at[idx_v.at[j]], buf, si).wait()

            def row_body(r, c2):
                for rr in range(2):
                    for k in range(D_MODEL // LANES):
                        sl = pl.ds(k * LANES, LANES)
                        buf[2 * r + rr, sl] = buf[2 * r + rr, sl] * SCALE
                return c2
            lax.fori_loop(0, C // 2, row_body, 0)

            pltpu.async_copy(buf, out_hbm.at[pl.ds(base + j * C, C)], so)

            if regather:
                @pl.when(j >= 1)
                def _():
                    pltpu.make_async_copy(
                        bufs[bg], out_hbm.at[pl.ds(base, C)], sos[bg]).wait()
                pltpu.async_copy(table_hbm.at[idx_v.at[j + 2]], bufs[bg],
                                 sis[bg])

        def outer(jj, carry):
            for u in range(RING):
                step(RING * jj + u, u, regather=True)
            return carry

        n_main = (chunks - 2) // RING
        lax.fori_loop(0, n_main, outer, 0)
        for j in range(n_main * RING, chunks):
            step(j, j % RING, regather=False)
        for j in range(chunks - RING, chunks):
            b = j % RING
            pltpu.make_async_copy(
                bufs[b], out_hbm.at[pl.ds(base + j * C, C)], sos[b]).wait()

    return emb_kernel(idx, table)


def kernel(x, table):
    b, s = x.shape
    B = b * s
    idx = x.reshape(NW, B // (NW * C), C).astype(jnp.int32)
    out = _emb(idx, table, B)
    return out.reshape(b, s, D_MODEL)
